# baseline (speedup 1.0000x reference)
import functools

import jax
import jax.numpy as jnp
from jax import lax
from jax.experimental import pallas as pl
from jax.experimental.pallas import tpu as pltpu
from jax.experimental.pallas import tpu_sc as plsc

_VOCAB = 1000000
_DIM = 32
_BATCH = 4096
_NNZ = 204800

_NW = 32
_SEG_PER_W = _BATCH // _NW
_CHUNK = 1024
_SUB = 128
_ACC_ROWS = _SEG_PER_W + 2

_NCOL = 7813
_BB = 84
_GB = 94
_NCOLP = _GB * _BB
_VPAD = _NCOL * 128


def _detile_body(in_ref, out_ref):
    x = in_ref[...]
    out_ref[...] = jnp.concatenate(
        [x[:, 128 * j:128 * (j + 1)] for j in range(_BB)], axis=0)


def _detile(params):
    return pl.pallas_call(
        _detile_body,
        grid=(4, _GB),
        in_specs=[pl.BlockSpec((8, _BB * 128), lambda a, g: (a, g))],
        out_specs=pl.BlockSpec((_BB * 8, 128), lambda a, g: (a * _GB + g, 0)),
        out_shape=jax.ShapeDtypeStruct((_NCOLP * 4 * 8, 128), jnp.float32),
    )(params.T)


def _shuffle_body(ts_hbm, out_hbm, tile_ref, row_ref, sem):
    wid = lax.axis_index("c") * 16 + lax.axis_index("s")
    nblk = jnp.where(wid < _NCOL - 244 * _NW, 245, 244)
    lane = lax.broadcasted_iota(jnp.int32, (16,), 0)

    def block(k, _):
        b = wid + k * _NW
        for a in range(4):
            pltpu.sync_copy(
                ts_hbm.at[pl.ds(pl.multiple_of((a * _NCOLP + b) * 8, 8), 8),
                          :],
                tile_ref.at[a])
        for g in range(8):
            idxv = (lane + g * 16) * _DIM
            for a in range(4):
                for p in range(8):
                    plsc.store_scatter(
                        row_ref, [idxv + 8 * a + p],
                        tile_ref[a, p, pl.ds(g * 16, 16)])
        pltpu.sync_copy(
            row_ref,
            out_hbm.at[pl.ds(b * 128 * _DIM, 128 * _DIM)])
        return 0

    lax.fori_loop(0, nblk, block, 0)


def _shuffle(ts):
    k = functools.partial(
        pl.kernel,
        out_type=jax.ShapeDtypeStruct((_VPAD * _DIM,), jnp.float32),
        mesh=plsc.VectorSubcoreMesh(core_axis_name="c", subcore_axis_name="s"),
        compiler_params=pltpu.CompilerParams(use_tc_tiling_on_sc=False,
                                             needs_layout_passes=False),
        scratch_types=[
            pltpu.VMEM((4, 8, 128), jnp.float32),
            pltpu.VMEM((128 * _DIM,), jnp.float32),
            pltpu.SemaphoreType.DMA,
        ],
    )(_shuffle_body)
    return k(ts)


@jax.jit
def _relayout(params):
    return _shuffle(_detile(params))


def _sel16(v, k):
    s = v[0]
    for j in range(1, 16):
        s = jnp.where(k == j, v[j], s)
    return s


def _lower_bound(seg_hbm, probe_ref, target):
    def body(_, carry):
        lo, hi = carry
        m = (lo + hi) // 2
        m8 = pl.multiple_of(jnp.minimum(m & ~7, _NNZ - 16), 8)
        pltpu.sync_copy(seg_hbm.at[pl.ds(m8, 16)], probe_ref)
        v = _sel16(probe_ref[pl.ds(0, 16)], m - m8)
        lt = v < target
        lo = jnp.where(lt, m + 1, lo)
        hi = jnp.where(lt, hi, m)
        return lo, hi

    lo, _ = lax.fori_loop(0, 18, body, (jnp.int32(0), jnp.int32(_NNZ)))
    return lo


def _gather_body(ids_hbm, seg_hbm, table_hbm, out_hbm,
                 probe_ref, idx_ref, segv_ref, rows_ref,
                 acc_ref, cnt_ref, sem):
    wid = lax.axis_index("c") * 16 + lax.axis_index("s")
    seg_base = wid * _SEG_PER_W
    lane = lax.broadcasted_iota(jnp.int32, (16,), 0)

    def zero_acc(k, _):
        acc_ref[pl.ds(k * 16, 16)] = jnp.zeros((16,), jnp.float32)
        return 0

    lax.fori_loop(0, (_ACC_ROWS * _DIM) // 16, zero_acc, 0)

    def zero_cnt(k, _):
        cnt_ref[k] = 0.0
        return 0

    lax.fori_loop(0, _ACC_ROWS, zero_cnt, 0)

    start = _lower_bound(seg_hbm, probe_ref, seg_base)
    end = _lower_bound(seg_hbm, probe_ref, seg_base + _SEG_PER_W)

    a0 = start & ~7
    e8 = (end + 7) & ~7
    nchunks = (e8 - a0 + _CHUNK - 1) // _CHUNK

    def chunk_body(t, _):
        logical = a0 + t * _CHUNK
        p = pl.multiple_of(
            jnp.minimum(logical, _NNZ - _CHUNK), 8)
        d = logical - p
        m = jnp.minimum(_CHUNK, e8 - logical)

        pltpu.sync_copy(ids_hbm.at[pl.ds(p, _CHUNK)], idx_ref)
        pltpu.sync_copy(seg_hbm.at[pl.ds(p, _CHUNK)], segv_ref)

        copies = []
        for j in range(_CHUNK // _SUB):
            copies.append(pltpu.make_async_copy(
                table_hbm.at[idx_ref.at[pl.ds(j * _SUB, _SUB)]],
                rows_ref.at[pl.ds(j * _SUB, _SUB), :],
                sem,
            ))
        for c in copies:
            c.start()
        for c in copies:
            c.wait()

        def accum(g, _):
            base = pl.multiple_of(g * 16, 16)
            sv = segv_ref[pl.ds(base, 16)]
            pos = base + lane
            ok = (pos >= d) & (pos < d + m)
            rv = jnp.clip(jnp.where(ok, sv - seg_base, -1), -1, _SEG_PER_W) + 1
            offv = rv * _DIM
            for j in range(16):
                off = offv[j]
                acc_ref[pl.ds(off, 16)] = (
                    acc_ref[pl.ds(off, 16)] + rows_ref[base + j, pl.ds(0, 16)])
                acc_ref[pl.ds(off + 16, 16)] = (
                    acc_ref[pl.ds(off + 16, 16)]
                    + rows_ref[base + j, pl.ds(16, 16)])
                r = rv[j]
                cnt_ref[r] = cnt_ref[r] + 1.0
            return 0

        lax.fori_loop(d // 16, (d + m + 15) // 16, accum, 0)
        return 0

    lax.fori_loop(0, nchunks, chunk_body, 0)

    def finalize(r, _):
        c = cnt_ref[r + 1]
        denom = jnp.maximum(jnp.full((16,), c, jnp.float32), 1.0)
        off = (r + 1) * _DIM
        acc_ref[pl.ds(off, 16)] = acc_ref[pl.ds(off, 16)] / denom
        acc_ref[pl.ds(off + 16, 16)] = acc_ref[pl.ds(off + 16, 16)] / denom
        return 0

    lax.fori_loop(0, _SEG_PER_W, finalize, 0)

    pltpu.sync_copy(acc_ref.at[pl.ds(_DIM, _SEG_PER_W * _DIM)],
                    out_hbm.at[pl.ds(seg_base * _DIM, _SEG_PER_W * _DIM)])


@jax.jit
def _run(ids, segment_ids, params):
    table = _relayout(params)
    k = functools.partial(
        pl.kernel,
        out_type=jax.ShapeDtypeStruct((_BATCH * _DIM,), jnp.float32),
        mesh=plsc.VectorSubcoreMesh(core_axis_name="c", subcore_axis_name="s"),
        compiler_params=pltpu.CompilerParams(use_tc_tiling_on_sc=False),
        scratch_types=[
            pltpu.VMEM((16,), jnp.int32),
            pltpu.VMEM((_CHUNK,), jnp.int32),
            pltpu.VMEM((_CHUNK,), jnp.int32),
            pltpu.VMEM((_CHUNK, _DIM), jnp.float32),
            pltpu.VMEM((_ACC_ROWS * _DIM,), jnp.float32),
            pltpu.SMEM((_ACC_ROWS,), jnp.float32),
            pltpu.SemaphoreType.DMA,
        ],
    )(_gather_body)
    out = k(ids, segment_ids, table.reshape(_VPAD, _DIM))
    return out.reshape(_BATCH, _DIM)


def kernel(ids, segment_ids, params):
    return _run(ids, segment_ids, params)


# speedup vs baseline: 1.6255x; 1.6255x over previous
import functools

import jax
import jax.numpy as jnp
from jax import lax
from jax.experimental import pallas as pl
from jax.experimental.pallas import tpu as pltpu
from jax.experimental.pallas import tpu_sc as plsc

_VOCAB = 1000000
_DIM = 32
_BATCH = 4096
_NNZ = 204800

_NW = 32
_SEG_PER_W = _BATCH // _NW
_CHUNK = 1024
_SUB = 128
_ACC_ROWS = _SEG_PER_W + 2

_NCOL = 7813
_BB = 84
_GB = 94
_NCOLP = _GB * _BB
_VPAD = _NCOL * 128


def _detile_body(in_ref, out_ref):
    x = in_ref[...]
    out_ref[...] = jnp.concatenate(
        [x[:, 128 * j:128 * (j + 1)] for j in range(_BB)], axis=0)


def _detile(params):
    return pl.pallas_call(
        _detile_body,
        grid=(4, _GB),
        in_specs=[pl.BlockSpec((8, _BB * 128), lambda a, g: (a, g))],
        out_specs=pl.BlockSpec((_BB * 8, 128), lambda a, g: (a * _GB + g, 0)),
        out_shape=jax.ShapeDtypeStruct((_NCOLP * 4 * 8, 128), jnp.float32),
    )(params.T)


def _shuffle_body(ts_hbm, out_hbm, tile_ref, row_ref, isem, osem):
    wid = lax.axis_index("c") * 16 + lax.axis_index("s")
    nblk = jnp.where(wid < _NCOL - 244 * _NW, 245, 244)
    lane = lax.broadcasted_iota(jnp.int32, (16,), 0)

    def fire_in(k, buf):
        b = wid + k * _NW
        for a in range(4):
            pltpu.make_async_copy(
                ts_hbm.at[pl.ds(pl.multiple_of((a * _NCOLP + b) * 8, 8), 8),
                          :],
                tile_ref.at[buf, a], isem).start()

    @pl.when(nblk > 0)
    def _():
        fire_in(0, 0)

    def block(k, _):
        b = wid + k * _NW
        buf = k % 2

        @pl.when(k + 1 < nblk)
        def _():
            fire_in(k + 1, 1 - buf)

        for a in range(4):
            pltpu.make_async_copy(
                ts_hbm.at[pl.ds(pl.multiple_of((a * _NCOLP + b) * 8, 8), 8),
                          :],
                tile_ref.at[buf, a], isem).wait()

        @pl.when(k >= 2)
        def _():
            b2 = wid + (k - 2) * _NW
            pltpu.make_async_copy(
                row_ref.at[buf],
                out_hbm.at[pl.ds(b2 * 128 * _DIM, 128 * _DIM)], osem).wait()

        for g in range(8):
            idxv = (lane + g * 16) * _DIM
            for a in range(4):
                for p in range(8):
                    plsc.store_scatter(
                        row_ref.at[buf], [idxv + 8 * a + p],
                        tile_ref[buf, a, p, pl.ds(g * 16, 16)])
        pltpu.make_async_copy(
            row_ref.at[buf],
            out_hbm.at[pl.ds(b * 128 * _DIM, 128 * _DIM)], osem).start()
        return 0

    lax.fori_loop(0, nblk, block, 0)

    def drain(k, _):
        @pl.when(k >= jnp.maximum(nblk - 2, 0))
        def _():
            b2 = wid + k * _NW
            pltpu.make_async_copy(
                row_ref.at[k % 2],
                out_hbm.at[pl.ds(b2 * 128 * _DIM, 128 * _DIM)], osem).wait()
        return 0

    lax.fori_loop(0, nblk, drain, 0)


def _shuffle(ts):
    k = functools.partial(
        pl.kernel,
        out_type=jax.ShapeDtypeStruct((_VPAD * _DIM,), jnp.float32),
        mesh=plsc.VectorSubcoreMesh(core_axis_name="c", subcore_axis_name="s"),
        compiler_params=pltpu.CompilerParams(use_tc_tiling_on_sc=False,
                                             needs_layout_passes=False),
        scratch_types=[
            pltpu.VMEM((2, 4, 8, 128), jnp.float32),
            pltpu.VMEM((2, 128 * _DIM), jnp.float32),
            pltpu.SemaphoreType.DMA,
            pltpu.SemaphoreType.DMA,
        ],
    )(_shuffle_body)
    return k(ts)


@jax.jit
def _relayout(params):
    return _shuffle(_detile(params))


def _sel16(v, k):
    s = v[0]
    for j in range(1, 16):
        s = jnp.where(k == j, v[j], s)
    return s


def _lower_bound(seg_hbm, probe_ref, target):
    def body(_, carry):
        lo, hi = carry
        m = (lo + hi) // 2
        m8 = pl.multiple_of(jnp.minimum(m & ~7, _NNZ - 16), 8)
        pltpu.sync_copy(seg_hbm.at[pl.ds(m8, 16)], probe_ref)
        v = _sel16(probe_ref[pl.ds(0, 16)], m - m8)
        lt = v < target
        lo = jnp.where(lt, m + 1, lo)
        hi = jnp.where(lt, hi, m)
        return lo, hi

    lo, _ = lax.fori_loop(0, 18, body, (jnp.int32(0), jnp.int32(_NNZ)))
    return lo


def _gather_body(ids_hbm, seg_hbm, table_hbm, out_hbm,
                 probe_ref, idx_ref, segv_ref, rows_ref,
                 acc_ref, cnt_ref, sem):
    wid = lax.axis_index("c") * 16 + lax.axis_index("s")
    seg_base = wid * _SEG_PER_W
    lane = lax.broadcasted_iota(jnp.int32, (16,), 0)

    def zero_acc(k, _):
        acc_ref[pl.ds(k * 16, 16)] = jnp.zeros((16,), jnp.float32)
        return 0

    lax.fori_loop(0, (_ACC_ROWS * _DIM) // 16, zero_acc, 0)

    def zero_cnt(k, _):
        cnt_ref[k] = 0.0
        return 0

    lax.fori_loop(0, _ACC_ROWS, zero_cnt, 0)

    start = _lower_bound(seg_hbm, probe_ref, seg_base)
    end = _lower_bound(seg_hbm, probe_ref, seg_base + _SEG_PER_W)

    a0 = start & ~7
    e8 = (end + 7) & ~7
    nchunks = (e8 - a0 + _CHUNK - 1) // _CHUNK

    def chunk_body(t, _):
        logical = a0 + t * _CHUNK
        p = pl.multiple_of(
            jnp.minimum(logical, _NNZ - _CHUNK), 8)
        d = logical - p
        m = jnp.minimum(_CHUNK, e8 - logical)

        pltpu.sync_copy(ids_hbm.at[pl.ds(p, _CHUNK)], idx_ref)
        pltpu.sync_copy(seg_hbm.at[pl.ds(p, _CHUNK)], segv_ref)

        copies = []
        for j in range(_CHUNK // _SUB):
            copies.append(pltpu.make_async_copy(
                table_hbm.at[idx_ref.at[pl.ds(j * _SUB, _SUB)]],
                rows_ref.at[pl.ds(j * _SUB, _SUB), :],
                sem,
            ))
        for c in copies:
            c.start()
        for c in copies:
            c.wait()

        def accum(g, _):
            base = pl.multiple_of(g * 16, 16)
            sv = segv_ref[pl.ds(base, 16)]
            pos = base + lane
            ok = (pos >= d) & (pos < d + m)
            rv = jnp.clip(jnp.where(ok, sv - seg_base, -1), -1, _SEG_PER_W) + 1
            offv = rv * _DIM
            for j in range(16):
                off = offv[j]
                acc_ref[pl.ds(off, 16)] = (
                    acc_ref[pl.ds(off, 16)] + rows_ref[base + j, pl.ds(0, 16)])
                acc_ref[pl.ds(off + 16, 16)] = (
                    acc_ref[pl.ds(off + 16, 16)]
                    + rows_ref[base + j, pl.ds(16, 16)])
                r = rv[j]
                cnt_ref[r] = cnt_ref[r] + 1.0
            return 0

        lax.fori_loop(d // 16, (d + m + 15) // 16, accum, 0)
        return 0

    lax.fori_loop(0, nchunks, chunk_body, 0)

    def finalize(r, _):
        c = cnt_ref[r + 1]
        denom = jnp.maximum(jnp.full((16,), c, jnp.float32), 1.0)
        off = (r + 1) * _DIM
        acc_ref[pl.ds(off, 16)] = acc_ref[pl.ds(off, 16)] / denom
        acc_ref[pl.ds(off + 16, 16)] = acc_ref[pl.ds(off + 16, 16)] / denom
        return 0

    lax.fori_loop(0, _SEG_PER_W, finalize, 0)

    pltpu.sync_copy(acc_ref.at[pl.ds(_DIM, _SEG_PER_W * _DIM)],
                    out_hbm.at[pl.ds(seg_base * _DIM, _SEG_PER_W * _DIM)])


@jax.jit
def _run(ids, segment_ids, params):
    table = _relayout(params)
    k = functools.partial(
        pl.kernel,
        out_type=jax.ShapeDtypeStruct((_BATCH * _DIM,), jnp.float32),
        mesh=plsc.VectorSubcoreMesh(core_axis_name="c", subcore_axis_name="s"),
        compiler_params=pltpu.CompilerParams(use_tc_tiling_on_sc=False),
        scratch_types=[
            pltpu.VMEM((16,), jnp.int32),
            pltpu.VMEM((_CHUNK,), jnp.int32),
            pltpu.VMEM((_CHUNK,), jnp.int32),
            pltpu.VMEM((_CHUNK, _DIM), jnp.float32),
            pltpu.VMEM((_ACC_ROWS * _DIM,), jnp.float32),
            pltpu.SMEM((_ACC_ROWS,), jnp.float32),
            pltpu.SemaphoreType.DMA,
        ],
    )(_gather_body)
    out = k(ids, segment_ids, table.reshape(_VPAD, _DIM))
    return out.reshape(_BATCH, _DIM)


def kernel(ids, segment_ids, params):
    return _run(ids, segment_ids, params)


# speedup vs baseline: 1.9036x; 1.1711x over previous
import functools

import jax
import jax.numpy as jnp
from jax import lax
from jax.experimental import pallas as pl
from jax.experimental.pallas import tpu as pltpu
from jax.experimental.pallas import tpu_sc as plsc

_VOCAB = 1000000
_DIM = 32
_BATCH = 4096
_NNZ = 204800

_NW = 32
_SEG_PER_W = _BATCH // _NW
_CHUNK = 1024
_SUB = 128
_ACC_ROWS = _SEG_PER_W + 2

_NCOL = 7813
_BB = 84
_GB = 94
_NCOLP = _GB * _BB
_VPAD = _NCOL * 128


def _detile_body(in_ref, out_ref):
    x = in_ref[...]
    out_ref[...] = jnp.concatenate(
        [x[:, 128 * j:128 * (j + 1)] for j in range(_BB)], axis=0)


def _detile(params):
    return pl.pallas_call(
        _detile_body,
        grid=(4, _GB),
        in_specs=[pl.BlockSpec((8, _BB * 128), lambda a, g: (a, g))],
        out_specs=pl.BlockSpec((_BB * 8, 128), lambda a, g: (a * _GB + g, 0)),
        out_shape=jax.ShapeDtypeStruct((_NCOLP * 4 * 8, 128), jnp.float32),
    )(params.T)


def _shuffle_body(ts_hbm, out_hbm, tile_ref, row_ref, isem, osem):
    wid = lax.axis_index("c") * 16 + lax.axis_index("s")
    nblk = jnp.where(wid < _NCOL - 244 * _NW, 245, 244)
    lane = lax.broadcasted_iota(jnp.int32, (16,), 0)

    def fire_in(k, buf):
        b = wid + k * _NW
        for a in range(4):
            pltpu.make_async_copy(
                ts_hbm.at[pl.ds(pl.multiple_of((a * _NCOLP + b) * 8, 8), 8),
                          :],
                tile_ref.at[buf, a], isem).start()

    @pl.when(nblk > 0)
    def _():
        fire_in(0, 0)

    def block(k, _):
        b = wid + k * _NW
        buf = k % 2

        @pl.when(k + 1 < nblk)
        def _():
            fire_in(k + 1, 1 - buf)

        for a in range(4):
            pltpu.make_async_copy(
                ts_hbm.at[pl.ds(pl.multiple_of((a * _NCOLP + b) * 8, 8), 8),
                          :],
                tile_ref.at[buf, a], isem).wait()

        @pl.when(k >= 2)
        def _():
            b2 = wid + (k - 2) * _NW
            pltpu.make_async_copy(
                row_ref.at[buf],
                out_hbm.at[pl.ds(b2 * 128 * _DIM, 128 * _DIM)], osem).wait()

        for g in range(8):
            idxv = (lane + g * 16) * _DIM
            vals = [tile_ref[buf, a, p, pl.ds(g * 16, 16)]
                    for a in range(4) for p in range(8)]
            for d in range(_DIM):
                plsc.store_scatter(row_ref.at[buf], [idxv + d], vals[d])
        pltpu.make_async_copy(
            row_ref.at[buf],
            out_hbm.at[pl.ds(b * 128 * _DIM, 128 * _DIM)], osem).start()
        return 0

    lax.fori_loop(0, nblk, block, 0)

    def drain(k, _):
        @pl.when(k >= jnp.maximum(nblk - 2, 0))
        def _():
            b2 = wid + k * _NW
            pltpu.make_async_copy(
                row_ref.at[k % 2],
                out_hbm.at[pl.ds(b2 * 128 * _DIM, 128 * _DIM)], osem).wait()
        return 0

    lax.fori_loop(0, nblk, drain, 0)


def _shuffle(ts):
    k = functools.partial(
        pl.kernel,
        out_type=jax.ShapeDtypeStruct((_VPAD * _DIM,), jnp.float32),
        mesh=plsc.VectorSubcoreMesh(core_axis_name="c", subcore_axis_name="s"),
        compiler_params=pltpu.CompilerParams(use_tc_tiling_on_sc=False,
                                             needs_layout_passes=False),
        scratch_types=[
            pltpu.VMEM((2, 4, 8, 128), jnp.float32),
            pltpu.VMEM((2, 128 * _DIM), jnp.float32),
            pltpu.SemaphoreType.DMA,
            pltpu.SemaphoreType.DMA,
        ],
    )(_shuffle_body)
    return k(ts)


@jax.jit
def _relayout(params):
    return _shuffle(_detile(params))


def _sel16(v, k):
    s = v[0]
    for j in range(1, 16):
        s = jnp.where(k == j, v[j], s)
    return s


def _lower_bound(seg_hbm, probe_ref, target):
    def body(_, carry):
        lo, hi = carry
        m = (lo + hi) // 2
        m8 = pl.multiple_of(jnp.minimum(m & ~7, _NNZ - 16), 8)
        pltpu.sync_copy(seg_hbm.at[pl.ds(m8, 16)], probe_ref)
        v = _sel16(probe_ref[pl.ds(0, 16)], m - m8)
        lt = v < target
        lo = jnp.where(lt, m + 1, lo)
        hi = jnp.where(lt, hi, m)
        return lo, hi

    lo, _ = lax.fori_loop(0, 18, body, (jnp.int32(0), jnp.int32(_NNZ)))
    return lo


def _gather_body(ids_hbm, seg_hbm, table_hbm, out_hbm,
                 probe_ref, idx_ref, segv_ref, rows_ref,
                 acc_ref, cnt_ref, sem):
    wid = lax.axis_index("c") * 16 + lax.axis_index("s")
    seg_base = wid * _SEG_PER_W
    lane = lax.broadcasted_iota(jnp.int32, (16,), 0)

    def zero_acc(k, _):
        acc_ref[pl.ds(k * 16, 16)] = jnp.zeros((16,), jnp.float32)
        return 0

    lax.fori_loop(0, (_ACC_ROWS * _DIM) // 16, zero_acc, 0)

    def zero_cnt(k, _):
        cnt_ref[k] = 0.0
        return 0

    lax.fori_loop(0, _ACC_ROWS, zero_cnt, 0)

    start = _lower_bound(seg_hbm, probe_ref, seg_base)
    end = _lower_bound(seg_hbm, probe_ref, seg_base + _SEG_PER_W)

    a0 = start & ~7
    e8 = (end + 7) & ~7
    nchunks = (e8 - a0 + _CHUNK - 1) // _CHUNK

    def chunk_body(t, _):
        logical = a0 + t * _CHUNK
        p = pl.multiple_of(
            jnp.minimum(logical, _NNZ - _CHUNK), 8)
        d = logical - p
        m = jnp.minimum(_CHUNK, e8 - logical)

        pltpu.sync_copy(ids_hbm.at[pl.ds(p, _CHUNK)], idx_ref)
        pltpu.sync_copy(seg_hbm.at[pl.ds(p, _CHUNK)], segv_ref)

        copies = []
        for j in range(_CHUNK // _SUB):
            copies.append(pltpu.make_async_copy(
                table_hbm.at[idx_ref.at[pl.ds(j * _SUB, _SUB)]],
                rows_ref.at[pl.ds(j * _SUB, _SUB), :],
                sem,
            ))
        for c in copies:
            c.start()
        for c in copies:
            c.wait()

        def accum(g, _):
            base = pl.multiple_of(g * 16, 16)
            sv = segv_ref[pl.ds(base, 16)]
            pos = base + lane
            ok = (pos >= d) & (pos < d + m)
            rv = jnp.clip(jnp.where(ok, sv - seg_base, -1), -1, _SEG_PER_W) + 1
            offv = rv * _DIM
            for j in range(16):
                off = offv[j]
                acc_ref[pl.ds(off, 16)] = (
                    acc_ref[pl.ds(off, 16)] + rows_ref[base + j, pl.ds(0, 16)])
                acc_ref[pl.ds(off + 16, 16)] = (
                    acc_ref[pl.ds(off + 16, 16)]
                    + rows_ref[base + j, pl.ds(16, 16)])
                r = rv[j]
                cnt_ref[r] = cnt_ref[r] + 1.0
            return 0

        lax.fori_loop(d // 16, (d + m + 15) // 16, accum, 0)
        return 0

    lax.fori_loop(0, nchunks, chunk_body, 0)

    def finalize(r, _):
        c = cnt_ref[r + 1]
        denom = jnp.maximum(jnp.full((16,), c, jnp.float32), 1.0)
        off = (r + 1) * _DIM
        acc_ref[pl.ds(off, 16)] = acc_ref[pl.ds(off, 16)] / denom
        acc_ref[pl.ds(off + 16, 16)] = acc_ref[pl.ds(off + 16, 16)] / denom
        return 0

    lax.fori_loop(0, _SEG_PER_W, finalize, 0)

    pltpu.sync_copy(acc_ref.at[pl.ds(_DIM, _SEG_PER_W * _DIM)],
                    out_hbm.at[pl.ds(seg_base * _DIM, _SEG_PER_W * _DIM)])


@jax.jit
def _run(ids, segment_ids, params):
    table = _relayout(params)
    k = functools.partial(
        pl.kernel,
        out_type=jax.ShapeDtypeStruct((_BATCH * _DIM,), jnp.float32),
        mesh=plsc.VectorSubcoreMesh(core_axis_name="c", subcore_axis_name="s"),
        compiler_params=pltpu.CompilerParams(use_tc_tiling_on_sc=False),
        scratch_types=[
            pltpu.VMEM((16,), jnp.int32),
            pltpu.VMEM((_CHUNK,), jnp.int32),
            pltpu.VMEM((_CHUNK,), jnp.int32),
            pltpu.VMEM((_CHUNK, _DIM), jnp.float32),
            pltpu.VMEM((_ACC_ROWS * _DIM,), jnp.float32),
            pltpu.SMEM((_ACC_ROWS,), jnp.float32),
            pltpu.SemaphoreType.DMA,
        ],
    )(_gather_body)
    out = k(ids, segment_ids, table.reshape(_VPAD, _DIM))
    return out.reshape(_BATCH, _DIM)


def kernel(ids, segment_ids, params):
    return _run(ids, segment_ids, params)


# speedup vs baseline: 2.9524x; 1.5510x over previous
import functools

import jax
import jax.numpy as jnp
from jax import lax
from jax.experimental import pallas as pl
from jax.experimental.pallas import tpu as pltpu
from jax.experimental.pallas import tpu_sc as plsc

_VOCAB = 1000000
_DIM = 32
_BATCH = 4096
_NNZ = 204800

_NW = 32
_SEG_PER_W = _BATCH // _NW
_CHUNK = 1024
_SUB = 128
_ACC_ROWS = _SEG_PER_W + 2

_NCOL = 7813
_BB = 84
_GB = 94
_NCOLP = _GB * _BB
_VPAD = _NCOL * 128


def _detile_body(in_ref, out_ref):
    x = in_ref[...]
    out_ref[...] = jnp.concatenate(
        [x[:, 128 * j:128 * (j + 1)] for j in range(_BB)], axis=0)


def _detile(params):
    return pl.pallas_call(
        _detile_body,
        grid=(4, _GB),
        in_specs=[pl.BlockSpec((8, _BB * 128), lambda a, g: (a, g))],
        out_specs=pl.BlockSpec((_BB * 8, 128), lambda a, g: (a * _GB + g, 0)),
        out_shape=jax.ShapeDtypeStruct((_NCOLP * 4 * 8, 128), jnp.float32),
    )(params.T)


def _shuffle_body(ts_hbm, out_hbm, tile_ref, row_ref, isem, osem):
    wid = lax.axis_index("c") * 16 + lax.axis_index("s")
    nblk = jnp.where(wid < _NCOL - 244 * _NW, 245, 244)
    lane = lax.broadcasted_iota(jnp.int32, (16,), 0)

    def fire_in(k, buf):
        b = wid + k * _NW
        for a in range(4):
            pltpu.make_async_copy(
                ts_hbm.at[pl.ds(pl.multiple_of((a * _NCOLP + b) * 8, 8), 8),
                          :],
                tile_ref.at[buf, pl.ds(8 * a, 8), pl.ds(0, 128)],
                isem).start()

    @pl.when(nblk > 0)
    def _():
        fire_in(0, 0)

    def block(k, _):
        b = wid + k * _NW
        buf = k % 2

        @pl.when(k + 1 < nblk)
        def _():
            fire_in(k + 1, 1 - buf)

        for a in range(4):
            pltpu.make_async_copy(
                ts_hbm.at[pl.ds(pl.multiple_of((a * _NCOLP + b) * 8, 8), 8),
                          :],
                tile_ref.at[buf, pl.ds(8 * a, 8), pl.ds(0, 128)],
                isem).wait()

        @pl.when(k >= 2)
        def _():
            b2 = wid + (k - 2) * _NW
            pltpu.make_async_copy(
                row_ref.at[buf],
                out_hbm.at[pl.ds(b2 * 128 * _DIM, 128 * _DIM)], osem).wait()

        def vgroup(g, _):
            for h in range(2):
                vs = [g * 16 + h * 8 + jj for jj in range(8)]
                los = [plsc.load_gather(
                    tile_ref.at[buf], [lane, jnp.full((16,), v, jnp.int32)])
                    for v in vs]
                his = [plsc.load_gather(
                    tile_ref.at[buf],
                    [lane + 16, jnp.full((16,), v, jnp.int32)])
                    for v in vs]
                for jj, v in enumerate(vs):
                    row_ref[buf, pl.ds(v * _DIM, 16)] = los[jj]
                    row_ref[buf, pl.ds(v * _DIM + 16, 16)] = his[jj]
            return 0

        lax.fori_loop(0, 8, vgroup, 0)
        pltpu.make_async_copy(
            row_ref.at[buf],
            out_hbm.at[pl.ds(b * 128 * _DIM, 128 * _DIM)], osem).start()
        return 0

    lax.fori_loop(0, nblk, block, 0)

    def drain(k, _):
        @pl.when(k >= jnp.maximum(nblk - 2, 0))
        def _():
            b2 = wid + k * _NW
            pltpu.make_async_copy(
                row_ref.at[k % 2],
                out_hbm.at[pl.ds(b2 * 128 * _DIM, 128 * _DIM)], osem).wait()
        return 0

    lax.fori_loop(0, nblk, drain, 0)


def _shuffle(ts):
    k = functools.partial(
        pl.kernel,
        out_type=jax.ShapeDtypeStruct((_VPAD * _DIM,), jnp.float32),
        mesh=plsc.VectorSubcoreMesh(core_axis_name="c", subcore_axis_name="s"),
        compiler_params=pltpu.CompilerParams(use_tc_tiling_on_sc=False,
                                             needs_layout_passes=False),
        scratch_types=[
            pltpu.VMEM((2, 32, 129), jnp.float32),
            pltpu.VMEM((2, 128 * _DIM), jnp.float32),
            pltpu.SemaphoreType.DMA,
            pltpu.SemaphoreType.DMA,
        ],
    )(_shuffle_body)
    return k(ts)


@jax.jit
def _relayout(params):
    return _shuffle(_detile(params))


def _sel16(v, k):
    s = v[0]
    for j in range(1, 16):
        s = jnp.where(k == j, v[j], s)
    return s


def _lower_bound(seg_hbm, probe_ref, target):
    def body(_, carry):
        lo, hi = carry
        m = (lo + hi) // 2
        m8 = pl.multiple_of(jnp.minimum(m & ~7, _NNZ - 16), 8)
        pltpu.sync_copy(seg_hbm.at[pl.ds(m8, 16)], probe_ref)
        v = _sel16(probe_ref[pl.ds(0, 16)], m - m8)
        lt = v < target
        lo = jnp.where(lt, m + 1, lo)
        hi = jnp.where(lt, hi, m)
        return lo, hi

    lo, _ = lax.fori_loop(0, 18, body, (jnp.int32(0), jnp.int32(_NNZ)))
    return lo


def _gather_body(ids_hbm, seg_hbm, table_hbm, out_hbm,
                 probe_ref, idx_ref, segv_ref, rows_ref,
                 acc_ref, cnt_ref, sem):
    wid = lax.axis_index("c") * 16 + lax.axis_index("s")
    seg_base = wid * _SEG_PER_W
    lane = lax.broadcasted_iota(jnp.int32, (16,), 0)

    def zero_acc(k, _):
        acc_ref[pl.ds(k * 16, 16)] = jnp.zeros((16,), jnp.float32)
        return 0

    lax.fori_loop(0, (_ACC_ROWS * _DIM) // 16, zero_acc, 0)

    def zero_cnt(k, _):
        cnt_ref[k] = 0.0
        return 0

    lax.fori_loop(0, _ACC_ROWS, zero_cnt, 0)

    start = _lower_bound(seg_hbm, probe_ref, seg_base)
    end = _lower_bound(seg_hbm, probe_ref, seg_base + _SEG_PER_W)

    a0 = start & ~7
    e8 = (end + 7) & ~7
    nchunks = (e8 - a0 + _CHUNK - 1) // _CHUNK

    def chunk_body(t, _):
        logical = a0 + t * _CHUNK
        p = pl.multiple_of(
            jnp.minimum(logical, _NNZ - _CHUNK), 8)
        d = logical - p
        m = jnp.minimum(_CHUNK, e8 - logical)

        pltpu.sync_copy(ids_hbm.at[pl.ds(p, _CHUNK)], idx_ref)
        pltpu.sync_copy(seg_hbm.at[pl.ds(p, _CHUNK)], segv_ref)

        copies = []
        for j in range(_CHUNK // _SUB):
            copies.append(pltpu.make_async_copy(
                table_hbm.at[idx_ref.at[pl.ds(j * _SUB, _SUB)]],
                rows_ref.at[pl.ds(j * _SUB, _SUB), :],
                sem,
            ))
        for c in copies:
            c.start()
        for c in copies:
            c.wait()

        def accum(g, _):
            base = pl.multiple_of(g * 16, 16)
            sv = segv_ref[pl.ds(base, 16)]
            pos = base + lane
            ok = (pos >= d) & (pos < d + m)
            rv = jnp.clip(jnp.where(ok, sv - seg_base, -1), -1, _SEG_PER_W) + 1
            offv = rv * _DIM
            for j in range(16):
                off = offv[j]
                acc_ref[pl.ds(off, 16)] = (
                    acc_ref[pl.ds(off, 16)] + rows_ref[base + j, pl.ds(0, 16)])
                acc_ref[pl.ds(off + 16, 16)] = (
                    acc_ref[pl.ds(off + 16, 16)]
                    + rows_ref[base + j, pl.ds(16, 16)])
                r = rv[j]
                cnt_ref[r] = cnt_ref[r] + 1.0
            return 0

        lax.fori_loop(d // 16, (d + m + 15) // 16, accum, 0)
        return 0

    lax.fori_loop(0, nchunks, chunk_body, 0)

    def finalize(r, _):
        c = cnt_ref[r + 1]
        denom = jnp.maximum(jnp.full((16,), c, jnp.float32), 1.0)
        off = (r + 1) * _DIM
        acc_ref[pl.ds(off, 16)] = acc_ref[pl.ds(off, 16)] / denom
        acc_ref[pl.ds(off + 16, 16)] = acc_ref[pl.ds(off + 16, 16)] / denom
        return 0

    lax.fori_loop(0, _SEG_PER_W, finalize, 0)

    pltpu.sync_copy(acc_ref.at[pl.ds(_DIM, _SEG_PER_W * _DIM)],
                    out_hbm.at[pl.ds(seg_base * _DIM, _SEG_PER_W * _DIM)])


@jax.jit
def _run(ids, segment_ids, params):
    table = _relayout(params)
    k = functools.partial(
        pl.kernel,
        out_type=jax.ShapeDtypeStruct((_BATCH * _DIM,), jnp.float32),
        mesh=plsc.VectorSubcoreMesh(core_axis_name="c", subcore_axis_name="s"),
        compiler_params=pltpu.CompilerParams(use_tc_tiling_on_sc=False),
        scratch_types=[
            pltpu.VMEM((16,), jnp.int32),
            pltpu.VMEM((_CHUNK,), jnp.int32),
            pltpu.VMEM((_CHUNK,), jnp.int32),
            pltpu.VMEM((_CHUNK, _DIM), jnp.float32),
            pltpu.VMEM((_ACC_ROWS * _DIM,), jnp.float32),
            pltpu.SMEM((_ACC_ROWS,), jnp.float32),
            pltpu.SemaphoreType.DMA,
        ],
    )(_gather_body)
    out = k(ids, segment_ids, table.reshape(_VPAD, _DIM))
    return out.reshape(_BATCH, _DIM)


def kernel(ids, segment_ids, params):
    return _run(ids, segment_ids, params)


# speedup vs baseline: 3.5425x; 1.1999x over previous
import functools

import jax
import jax.numpy as jnp
from jax import lax
from jax.experimental import pallas as pl
from jax.experimental.pallas import tpu as pltpu
from jax.experimental.pallas import tpu_sc as plsc

_VOCAB = 1000000
_DIM = 32
_BATCH = 4096
_NNZ = 204800

_NW = 32
_SEG_PER_W = _BATCH // _NW
_CHUNK = 1024
_SUB = 128
_ACC_ROWS = _SEG_PER_W + 2

_NCOL = 7813
_BB = 168
_GB = 47
_NCOLP = _GB * _BB
_VPAD = _NCOL * 128


def _detile_body(in_ref, out_ref):
    x = in_ref[...]
    out_ref[...] = jnp.concatenate(
        [x[:, 128 * j:128 * (j + 1)] for j in range(_BB)], axis=0)


def _detile(params):
    return pl.pallas_call(
        _detile_body,
        grid=(4, _GB),
        in_specs=[pl.BlockSpec((8, _BB * 128), lambda a, g: (a, g))],
        out_specs=pl.BlockSpec((_BB * 8, 128), lambda a, g: (a * _GB + g, 0)),
        out_shape=jax.ShapeDtypeStruct((_NCOLP * 4 * 8, 128), jnp.float32),
    )(params.T)


def _shuffle_body(ts_hbm, out_hbm, tile_ref, row_ref, isem, osem):
    wid = lax.axis_index("c") * 16 + lax.axis_index("s")
    nblk = jnp.where(wid < _NCOL - 244 * _NW, 245, 244)
    lane = lax.broadcasted_iota(jnp.int32, (16,), 0)

    def fire_in(k, buf):
        b = wid + k * _NW
        for a in range(4):
            pltpu.make_async_copy(
                ts_hbm.at[pl.ds(pl.multiple_of((a * _NCOLP + b) * 8, 8), 8),
                          :],
                tile_ref.at[buf, pl.ds(8 * a, 8), pl.ds(0, 128)],
                isem).start()

    @pl.when(nblk > 0)
    def _():
        fire_in(0, 0)

    def block(k, _):
        b = wid + k * _NW
        buf = k % 2

        @pl.when(k + 1 < nblk)
        def _():
            fire_in(k + 1, 1 - buf)

        for a in range(4):
            pltpu.make_async_copy(
                ts_hbm.at[pl.ds(pl.multiple_of((a * _NCOLP + b) * 8, 8), 8),
                          :],
                tile_ref.at[buf, pl.ds(8 * a, 8), pl.ds(0, 128)],
                isem).wait()

        @pl.when(k >= 2)
        def _():
            b2 = wid + (k - 2) * _NW
            pltpu.make_async_copy(
                row_ref.at[buf],
                out_hbm.at[pl.ds(b2 * 128 * _DIM, 128 * _DIM)], osem).wait()

        def vgroup(g, _):
            for h in range(2):
                vs = [g * 16 + h * 8 + jj for jj in range(8)]
                los = [plsc.load_gather(
                    tile_ref.at[buf], [lane, jnp.full((16,), v, jnp.int32)])
                    for v in vs]
                his = [plsc.load_gather(
                    tile_ref.at[buf],
                    [lane + 16, jnp.full((16,), v, jnp.int32)])
                    for v in vs]
                for jj, v in enumerate(vs):
                    row_ref[buf, pl.ds(v * _DIM, 16)] = los[jj]
                    row_ref[buf, pl.ds(v * _DIM + 16, 16)] = his[jj]
            return 0

        lax.fori_loop(0, 8, vgroup, 0)
        pltpu.make_async_copy(
            row_ref.at[buf],
            out_hbm.at[pl.ds(b * 128 * _DIM, 128 * _DIM)], osem).start()
        return 0

    lax.fori_loop(0, nblk, block, 0)

    def drain(k, _):
        @pl.when(k >= jnp.maximum(nblk - 2, 0))
        def _():
            b2 = wid + k * _NW
            pltpu.make_async_copy(
                row_ref.at[k % 2],
                out_hbm.at[pl.ds(b2 * 128 * _DIM, 128 * _DIM)], osem).wait()
        return 0

    lax.fori_loop(0, nblk, drain, 0)


def _shuffle(ts):
    k = functools.partial(
        pl.kernel,
        out_type=jax.ShapeDtypeStruct((_VPAD * _DIM,), jnp.float32),
        mesh=plsc.VectorSubcoreMesh(core_axis_name="c", subcore_axis_name="s"),
        compiler_params=pltpu.CompilerParams(use_tc_tiling_on_sc=False,
                                             needs_layout_passes=False),
        scratch_types=[
            pltpu.VMEM((2, 32, 129), jnp.float32),
            pltpu.VMEM((2, 128 * _DIM), jnp.float32),
            pltpu.SemaphoreType.DMA,
            pltpu.SemaphoreType.DMA,
        ],
    )(_shuffle_body)
    return k(ts)


@jax.jit
def _relayout(params):
    return _shuffle(_detile(params))


def _sel16(v, k):
    s = v[0]
    for j in range(1, 16):
        s = jnp.where(k == j, v[j], s)
    return s


def _lower_bound(seg_hbm, probe_ref, target):
    def body(_, carry):
        lo, hi = carry
        m = (lo + hi) // 2
        m8 = pl.multiple_of(jnp.minimum(m & ~7, _NNZ - 16), 8)
        pltpu.sync_copy(seg_hbm.at[pl.ds(m8, 16)], probe_ref)
        v = _sel16(probe_ref[pl.ds(0, 16)], m - m8)
        lt = v < target
        lo = jnp.where(lt, m + 1, lo)
        hi = jnp.where(lt, hi, m)
        return lo, hi

    lo, _ = lax.fori_loop(0, 18, body, (jnp.int32(0), jnp.int32(_NNZ)))
    return lo


def _gather_body(ids_hbm, seg_hbm, table_hbm, out_hbm,
                 probe_ref, idx_ref, segv_ref, rows_ref,
                 acc_ref, cnt_ref, sem):
    wid = lax.axis_index("c") * 16 + lax.axis_index("s")
    seg_base = wid * _SEG_PER_W
    lane = lax.broadcasted_iota(jnp.int32, (16,), 0)

    def zero_acc(k, _):
        acc_ref[pl.ds(k * 16, 16)] = jnp.zeros((16,), jnp.float32)
        return 0

    lax.fori_loop(0, (_ACC_ROWS * _DIM) // 16, zero_acc, 0)

    def zero_cnt(k, _):
        cnt_ref[k] = 0.0
        return 0

    lax.fori_loop(0, _ACC_ROWS, zero_cnt, 0)

    start = _lower_bound(seg_hbm, probe_ref, seg_base)
    end = _lower_bound(seg_hbm, probe_ref, seg_base + _SEG_PER_W)

    a0 = start & ~7
    e8 = (end + 7) & ~7
    nchunks = (e8 - a0 + _CHUNK - 1) // _CHUNK

    def chunk_body(t, _):
        logical = a0 + t * _CHUNK
        p = pl.multiple_of(
            jnp.minimum(logical, _NNZ - _CHUNK), 8)
        d = logical - p
        m = jnp.minimum(_CHUNK, e8 - logical)

        pltpu.sync_copy(ids_hbm.at[pl.ds(p, _CHUNK)], idx_ref)
        pltpu.sync_copy(seg_hbm.at[pl.ds(p, _CHUNK)], segv_ref)

        copies = []
        for j in range(_CHUNK // _SUB):
            copies.append(pltpu.make_async_copy(
                table_hbm.at[idx_ref.at[pl.ds(j * _SUB, _SUB)]],
                rows_ref.at[pl.ds(j * _SUB, _SUB), :],
                sem,
            ))
        for c in copies:
            c.start()
        for c in copies:
            c.wait()

        def accum(g, _):
            base = pl.multiple_of(g * 16, 16)
            sv = segv_ref[pl.ds(base, 16)]
            pos = base + lane
            ok = (pos >= d) & (pos < d + m)
            rv = jnp.clip(jnp.where(ok, sv - seg_base, -1), -1, _SEG_PER_W) + 1
            offv = rv * _DIM
            for j in range(16):
                off = offv[j]
                acc_ref[pl.ds(off, 16)] = (
                    acc_ref[pl.ds(off, 16)] + rows_ref[base + j, pl.ds(0, 16)])
                acc_ref[pl.ds(off + 16, 16)] = (
                    acc_ref[pl.ds(off + 16, 16)]
                    + rows_ref[base + j, pl.ds(16, 16)])
                r = rv[j]
                cnt_ref[r] = cnt_ref[r] + 1.0
            return 0

        lax.fori_loop(d // 16, (d + m + 15) // 16, accum, 0)
        return 0

    lax.fori_loop(0, nchunks, chunk_body, 0)

    def finalize(r, _):
        c = cnt_ref[r + 1]
        denom = jnp.maximum(jnp.full((16,), c, jnp.float32), 1.0)
        off = (r + 1) * _DIM
        acc_ref[pl.ds(off, 16)] = acc_ref[pl.ds(off, 16)] / denom
        acc_ref[pl.ds(off + 16, 16)] = acc_ref[pl.ds(off + 16, 16)] / denom
        return 0

    lax.fori_loop(0, _SEG_PER_W, finalize, 0)

    pltpu.sync_copy(acc_ref.at[pl.ds(_DIM, _SEG_PER_W * _DIM)],
                    out_hbm.at[pl.ds(seg_base * _DIM, _SEG_PER_W * _DIM)])


@jax.jit
def _run(ids, segment_ids, params):
    table = _relayout(params)
    k = functools.partial(
        pl.kernel,
        out_type=jax.ShapeDtypeStruct((_BATCH * _DIM,), jnp.float32),
        mesh=plsc.VectorSubcoreMesh(core_axis_name="c", subcore_axis_name="s"),
        compiler_params=pltpu.CompilerParams(use_tc_tiling_on_sc=False),
        scratch_types=[
            pltpu.VMEM((16,), jnp.int32),
            pltpu.VMEM((_CHUNK,), jnp.int32),
            pltpu.VMEM((_CHUNK,), jnp.int32),
            pltpu.VMEM((_CHUNK, _DIM), jnp.float32),
            pltpu.VMEM((_ACC_ROWS * _DIM,), jnp.float32),
            pltpu.SMEM((_ACC_ROWS,), jnp.float32),
            pltpu.SemaphoreType.DMA,
        ],
    )(_gather_body)
    out = k(ids, segment_ids, table.reshape(_VPAD, _DIM))
    return out.reshape(_BATCH, _DIM)


def kernel(ids, segment_ids, params):
    return _run(ids, segment_ids, params)


# speedup vs baseline: 3.9201x; 1.1066x over previous
import functools

import jax
import jax.numpy as jnp
from jax import lax
from jax.experimental import pallas as pl
from jax.experimental.pallas import tpu as pltpu
from jax.experimental.pallas import tpu_sc as plsc

_VOCAB = 1000000
_DIM = 32
_BATCH = 4096
_NNZ = 204800

_NW = 32
_SEG_PER_W = _BATCH // _NW
_CHUNK = 1024
_SUB = 128
_ACC_ROWS = _SEG_PER_W + 2

_NCOL = 7813
_BB = 336
_GB = 24
_NCOLP = _GB * _BB
_VPAD = _NCOL * 128


def _detile_body(in_ref, out_ref):
    x = in_ref[...]
    out_ref[...] = jnp.concatenate(
        [x[:, 128 * j:128 * (j + 1)] for j in range(_BB)], axis=0)


def _detile(params):
    return pl.pallas_call(
        _detile_body,
        grid=(4, _GB),
        in_specs=[pl.BlockSpec((8, _BB * 128), lambda a, g: (a, g))],
        out_specs=pl.BlockSpec((_BB * 8, 128), lambda a, g: (a * _GB + g, 0)),
        out_shape=jax.ShapeDtypeStruct((_NCOLP * 4 * 8, 128), jnp.float32),
    )(params.T)


def _shuffle_body(ts_hbm, out_hbm, tile_ref, row_ref, isem, osem):
    wid = lax.axis_index("c") * 16 + lax.axis_index("s")
    nblk = jnp.where(wid < _NCOL - 244 * _NW, 245, 244)
    lane = lax.broadcasted_iota(jnp.int32, (16,), 0)

    def fire_in(k, buf):
        b = wid + k * _NW
        for a in range(4):
            pltpu.make_async_copy(
                ts_hbm.at[pl.ds(pl.multiple_of((a * _NCOLP + b) * 8, 8), 8),
                          :],
                tile_ref.at[buf, pl.ds(8 * a, 8), pl.ds(0, 128)],
                isem).start()

    @pl.when(nblk > 0)
    def _():
        fire_in(0, 0)

    def block(k, _):
        b = wid + k * _NW
        buf = k % 2

        @pl.when(k + 1 < nblk)
        def _():
            fire_in(k + 1, 1 - buf)

        for a in range(4):
            pltpu.make_async_copy(
                ts_hbm.at[pl.ds(pl.multiple_of((a * _NCOLP + b) * 8, 8), 8),
                          :],
                tile_ref.at[buf, pl.ds(8 * a, 8), pl.ds(0, 128)],
                isem).wait()

        @pl.when(k >= 2)
        def _():
            b2 = wid + (k - 2) * _NW
            pltpu.make_async_copy(
                row_ref.at[buf],
                out_hbm.at[pl.ds(b2 * 128 * _DIM, 128 * _DIM)], osem).wait()

        def vgroup(g, _):
            for h in range(2):
                vs = [g * 16 + h * 8 + jj for jj in range(8)]
                los = [plsc.load_gather(
                    tile_ref.at[buf], [lane, jnp.full((16,), v, jnp.int32)])
                    for v in vs]
                his = [plsc.load_gather(
                    tile_ref.at[buf],
                    [lane + 16, jnp.full((16,), v, jnp.int32)])
                    for v in vs]
                for jj, v in enumerate(vs):
                    row_ref[buf, pl.ds(v * _DIM, 16)] = los[jj]
                    row_ref[buf, pl.ds(v * _DIM + 16, 16)] = his[jj]
            return 0

        lax.fori_loop(0, 8, vgroup, 0)
        pltpu.make_async_copy(
            row_ref.at[buf],
            out_hbm.at[pl.ds(b * 128 * _DIM, 128 * _DIM)], osem).start()
        return 0

    lax.fori_loop(0, nblk, block, 0)

    def drain(k, _):
        @pl.when(k >= jnp.maximum(nblk - 2, 0))
        def _():
            b2 = wid + k * _NW
            pltpu.make_async_copy(
                row_ref.at[k % 2],
                out_hbm.at[pl.ds(b2 * 128 * _DIM, 128 * _DIM)], osem).wait()
        return 0

    lax.fori_loop(0, nblk, drain, 0)


def _shuffle(ts):
    k = functools.partial(
        pl.kernel,
        out_type=jax.ShapeDtypeStruct((_VPAD * _DIM,), jnp.float32),
        mesh=plsc.VectorSubcoreMesh(core_axis_name="c", subcore_axis_name="s"),
        compiler_params=pltpu.CompilerParams(use_tc_tiling_on_sc=False,
                                             needs_layout_passes=False),
        scratch_types=[
            pltpu.VMEM((2, 32, 129), jnp.float32),
            pltpu.VMEM((2, 128 * _DIM), jnp.float32),
            pltpu.SemaphoreType.DMA,
            pltpu.SemaphoreType.DMA,
        ],
    )(_shuffle_body)
    return k(ts)


@jax.jit
def _relayout(params):
    return _shuffle(_detile(params))


def _sel16(v, k):
    s = v[0]
    for j in range(1, 16):
        s = jnp.where(k == j, v[j], s)
    return s


def _lower_bound(seg_hbm, probe_ref, target):
    def body(_, carry):
        lo, hi = carry
        m = (lo + hi) // 2
        m8 = pl.multiple_of(jnp.minimum(m & ~7, _NNZ - 16), 8)
        pltpu.sync_copy(seg_hbm.at[pl.ds(m8, 16)], probe_ref)
        v = _sel16(probe_ref[pl.ds(0, 16)], m - m8)
        lt = v < target
        lo = jnp.where(lt, m + 1, lo)
        hi = jnp.where(lt, hi, m)
        return lo, hi

    lo, _ = lax.fori_loop(0, 18, body, (jnp.int32(0), jnp.int32(_NNZ)))
    return lo


def _gather_body(ids_hbm, seg_hbm, table_hbm, out_hbm,
                 probe_ref, idx_ref, segv_ref, rows_ref,
                 acc_ref, cnt_ref, sem):
    wid = lax.axis_index("c") * 16 + lax.axis_index("s")
    seg_base = wid * _SEG_PER_W
    lane = lax.broadcasted_iota(jnp.int32, (16,), 0)

    def zero_acc(k, _):
        acc_ref[pl.ds(k * 16, 16)] = jnp.zeros((16,), jnp.float32)
        return 0

    lax.fori_loop(0, (_ACC_ROWS * _DIM) // 16, zero_acc, 0)

    def zero_cnt(k, _):
        cnt_ref[k] = 0.0
        return 0

    lax.fori_loop(0, _ACC_ROWS, zero_cnt, 0)

    start = _lower_bound(seg_hbm, probe_ref, seg_base)
    end = _lower_bound(seg_hbm, probe_ref, seg_base + _SEG_PER_W)

    a0 = start & ~7
    e8 = (end + 7) & ~7
    nchunks = (e8 - a0 + _CHUNK - 1) // _CHUNK

    def chunk_body(t, _):
        logical = a0 + t * _CHUNK
        p = pl.multiple_of(
            jnp.minimum(logical, _NNZ - _CHUNK), 8)
        d = logical - p
        m = jnp.minimum(_CHUNK, e8 - logical)

        pltpu.sync_copy(ids_hbm.at[pl.ds(p, _CHUNK)], idx_ref)
        pltpu.sync_copy(seg_hbm.at[pl.ds(p, _CHUNK)], segv_ref)

        copies = []
        for j in range(_CHUNK // _SUB):
            copies.append(pltpu.make_async_copy(
                table_hbm.at[idx_ref.at[pl.ds(j * _SUB, _SUB)]],
                rows_ref.at[pl.ds(j * _SUB, _SUB), :],
                sem,
            ))
        for c in copies:
            c.start()
        for c in copies:
            c.wait()

        def accum(g, _):
            base = pl.multiple_of(g * 16, 16)
            sv = segv_ref[pl.ds(base, 16)]
            pos = base + lane
            ok = (pos >= d) & (pos < d + m)
            rv = jnp.clip(jnp.where(ok, sv - seg_base, -1), -1, _SEG_PER_W) + 1
            offv = rv * _DIM
            for j in range(16):
                off = offv[j]
                acc_ref[pl.ds(off, 16)] = (
                    acc_ref[pl.ds(off, 16)] + rows_ref[base + j, pl.ds(0, 16)])
                acc_ref[pl.ds(off + 16, 16)] = (
                    acc_ref[pl.ds(off + 16, 16)]
                    + rows_ref[base + j, pl.ds(16, 16)])
                r = rv[j]
                cnt_ref[r] = cnt_ref[r] + 1.0
            return 0

        lax.fori_loop(d // 16, (d + m + 15) // 16, accum, 0)
        return 0

    lax.fori_loop(0, nchunks, chunk_body, 0)

    def finalize(r, _):
        c = cnt_ref[r + 1]
        denom = jnp.maximum(jnp.full((16,), c, jnp.float32), 1.0)
        off = (r + 1) * _DIM
        acc_ref[pl.ds(off, 16)] = acc_ref[pl.ds(off, 16)] / denom
        acc_ref[pl.ds(off + 16, 16)] = acc_ref[pl.ds(off + 16, 16)] / denom
        return 0

    lax.fori_loop(0, _SEG_PER_W, finalize, 0)

    pltpu.sync_copy(acc_ref.at[pl.ds(_DIM, _SEG_PER_W * _DIM)],
                    out_hbm.at[pl.ds(seg_base * _DIM, _SEG_PER_W * _DIM)])


@jax.jit
def _run(ids, segment_ids, params):
    table = _relayout(params)
    k = functools.partial(
        pl.kernel,
        out_type=jax.ShapeDtypeStruct((_BATCH * _DIM,), jnp.float32),
        mesh=plsc.VectorSubcoreMesh(core_axis_name="c", subcore_axis_name="s"),
        compiler_params=pltpu.CompilerParams(use_tc_tiling_on_sc=False),
        scratch_types=[
            pltpu.VMEM((16,), jnp.int32),
            pltpu.VMEM((_CHUNK,), jnp.int32),
            pltpu.VMEM((_CHUNK,), jnp.int32),
            pltpu.VMEM((_CHUNK, _DIM), jnp.float32),
            pltpu.VMEM((_ACC_ROWS * _DIM,), jnp.float32),
            pltpu.SMEM((_ACC_ROWS,), jnp.float32),
            pltpu.SemaphoreType.DMA,
        ],
    )(_gather_body)
    out = k(ids, segment_ids, table.reshape(_VPAD, _DIM))
    return out.reshape(_BATCH, _DIM)


def kernel(ids, segment_ids, params):
    return _run(ids, segment_ids, params)


# speedup vs baseline: 4.1566x; 1.0603x over previous
import functools

import jax
import jax.numpy as jnp
from jax import lax
from jax.experimental import pallas as pl
from jax.experimental.pallas import tpu as pltpu
from jax.experimental.pallas import tpu_sc as plsc

_VOCAB = 1000000
_DIM = 32
_BATCH = 4096
_NNZ = 204800

_NW = 32
_SEG_PER_W = _BATCH // _NW
_CHUNK = 1024
_SUB = 128
_ACC_ROWS = _SEG_PER_W + 2

_NCOL = 7813
_BB = 672
_GB = 12
_NCOLP = _GB * _BB
_VPAD = _NCOL * 128


def _detile_body(in_ref, out_ref):
    x = in_ref[...]
    out_ref[...] = jnp.concatenate(
        [x[:, 128 * j:128 * (j + 1)] for j in range(_BB)], axis=0)


def _detile(params):
    return pl.pallas_call(
        _detile_body,
        grid=(4, _GB),
        in_specs=[pl.BlockSpec((8, _BB * 128), lambda a, g: (a, g))],
        out_specs=pl.BlockSpec((_BB * 8, 128), lambda a, g: (a * _GB + g, 0)),
        out_shape=jax.ShapeDtypeStruct((_NCOLP * 4 * 8, 128), jnp.float32),
    )(params.T)


def _shuffle_body(ts_hbm, out_hbm, tile_ref, row_ref, isem, osem):
    wid = lax.axis_index("c") * 16 + lax.axis_index("s")
    nblk = jnp.where(wid < _NCOL - 244 * _NW, 245, 244)
    lane = lax.broadcasted_iota(jnp.int32, (16,), 0)

    def fire_in(k, buf):
        b = wid + k * _NW
        for a in range(4):
            pltpu.make_async_copy(
                ts_hbm.at[pl.ds(pl.multiple_of((a * _NCOLP + b) * 8, 8), 8),
                          :],
                tile_ref.at[buf, pl.ds(8 * a, 8), pl.ds(0, 128)],
                isem).start()

    @pl.when(nblk > 0)
    def _():
        fire_in(0, 0)

    def block(k, _):
        b = wid + k * _NW
        buf = k % 2

        @pl.when(k + 1 < nblk)
        def _():
            fire_in(k + 1, 1 - buf)

        for a in range(4):
            pltpu.make_async_copy(
                ts_hbm.at[pl.ds(pl.multiple_of((a * _NCOLP + b) * 8, 8), 8),
                          :],
                tile_ref.at[buf, pl.ds(8 * a, 8), pl.ds(0, 128)],
                isem).wait()

        @pl.when(k >= 2)
        def _():
            b2 = wid + (k - 2) * _NW
            pltpu.make_async_copy(
                row_ref.at[buf],
                out_hbm.at[pl.ds(b2 * 128 * _DIM, 128 * _DIM)], osem).wait()

        def vgroup(g, _):
            for h in range(2):
                vs = [g * 16 + h * 8 + jj for jj in range(8)]
                los = [plsc.load_gather(
                    tile_ref.at[buf], [lane, jnp.full((16,), v, jnp.int32)])
                    for v in vs]
                his = [plsc.load_gather(
                    tile_ref.at[buf],
                    [lane + 16, jnp.full((16,), v, jnp.int32)])
                    for v in vs]
                for jj, v in enumerate(vs):
                    row_ref[buf, pl.ds(v * _DIM, 16)] = los[jj]
                    row_ref[buf, pl.ds(v * _DIM + 16, 16)] = his[jj]
            return 0

        lax.fori_loop(0, 8, vgroup, 0)
        pltpu.make_async_copy(
            row_ref.at[buf],
            out_hbm.at[pl.ds(b * 128 * _DIM, 128 * _DIM)], osem).start()
        return 0

    lax.fori_loop(0, nblk, block, 0)

    def drain(k, _):
        @pl.when(k >= jnp.maximum(nblk - 2, 0))
        def _():
            b2 = wid + k * _NW
            pltpu.make_async_copy(
                row_ref.at[k % 2],
                out_hbm.at[pl.ds(b2 * 128 * _DIM, 128 * _DIM)], osem).wait()
        return 0

    lax.fori_loop(0, nblk, drain, 0)


def _shuffle(ts):
    k = functools.partial(
        pl.kernel,
        out_type=jax.ShapeDtypeStruct((_VPAD * _DIM,), jnp.float32),
        mesh=plsc.VectorSubcoreMesh(core_axis_name="c", subcore_axis_name="s"),
        compiler_params=pltpu.CompilerParams(use_tc_tiling_on_sc=False,
                                             needs_layout_passes=False),
        scratch_types=[
            pltpu.VMEM((2, 32, 129), jnp.float32),
            pltpu.VMEM((2, 128 * _DIM), jnp.float32),
            pltpu.SemaphoreType.DMA,
            pltpu.SemaphoreType.DMA,
        ],
    )(_shuffle_body)
    return k(ts)


@jax.jit
def _relayout(params):
    return _shuffle(_detile(params))


def _sel16(v, k):
    s = v[0]
    for j in range(1, 16):
        s = jnp.where(k == j, v[j], s)
    return s


def _lower_bound(seg_hbm, probe_ref, target):
    def body(_, carry):
        lo, hi = carry
        m = (lo + hi) // 2
        m8 = pl.multiple_of(jnp.minimum(m & ~7, _NNZ - 16), 8)
        pltpu.sync_copy(seg_hbm.at[pl.ds(m8, 16)], probe_ref)
        v = _sel16(probe_ref[pl.ds(0, 16)], m - m8)
        lt = v < target
        lo = jnp.where(lt, m + 1, lo)
        hi = jnp.where(lt, hi, m)
        return lo, hi

    lo, _ = lax.fori_loop(0, 18, body, (jnp.int32(0), jnp.int32(_NNZ)))
    return lo


def _gather_body(ids_hbm, seg_hbm, table_hbm, out_hbm,
                 probe_ref, idx_ref, segv_ref, rows_ref,
                 acc_ref, cnt_ref, sem):
    wid = lax.axis_index("c") * 16 + lax.axis_index("s")
    seg_base = wid * _SEG_PER_W
    lane = lax.broadcasted_iota(jnp.int32, (16,), 0)

    def zero_acc(k, _):
        acc_ref[pl.ds(k * 16, 16)] = jnp.zeros((16,), jnp.float32)
        return 0

    lax.fori_loop(0, (_ACC_ROWS * _DIM) // 16, zero_acc, 0)

    def zero_cnt(k, _):
        cnt_ref[k] = 0.0
        return 0

    lax.fori_loop(0, _ACC_ROWS, zero_cnt, 0)

    start = _lower_bound(seg_hbm, probe_ref, seg_base)
    end = _lower_bound(seg_hbm, probe_ref, seg_base + _SEG_PER_W)

    a0 = start & ~7
    e8 = (end + 7) & ~7
    nchunks = (e8 - a0 + _CHUNK - 1) // _CHUNK

    def chunk_body(t, _):
        logical = a0 + t * _CHUNK
        p = pl.multiple_of(
            jnp.minimum(logical, _NNZ - _CHUNK), 8)
        d = logical - p
        m = jnp.minimum(_CHUNK, e8 - logical)

        pltpu.sync_copy(ids_hbm.at[pl.ds(p, _CHUNK)], idx_ref)
        pltpu.sync_copy(seg_hbm.at[pl.ds(p, _CHUNK)], segv_ref)

        copies = []
        for j in range(_CHUNK // _SUB):
            copies.append(pltpu.make_async_copy(
                table_hbm.at[idx_ref.at[pl.ds(j * _SUB, _SUB)]],
                rows_ref.at[pl.ds(j * _SUB, _SUB), :],
                sem,
            ))
        for c in copies:
            c.start()
        for c in copies:
            c.wait()

        def accum(g, _):
            base = pl.multiple_of(g * 16, 16)
            sv = segv_ref[pl.ds(base, 16)]
            pos = base + lane
            ok = (pos >= d) & (pos < d + m)
            rv = jnp.clip(jnp.where(ok, sv - seg_base, -1), -1, _SEG_PER_W) + 1
            offv = rv * _DIM
            for j in range(16):
                off = offv[j]
                acc_ref[pl.ds(off, 16)] = (
                    acc_ref[pl.ds(off, 16)] + rows_ref[base + j, pl.ds(0, 16)])
                acc_ref[pl.ds(off + 16, 16)] = (
                    acc_ref[pl.ds(off + 16, 16)]
                    + rows_ref[base + j, pl.ds(16, 16)])
                r = rv[j]
                cnt_ref[r] = cnt_ref[r] + 1.0
            return 0

        lax.fori_loop(d // 16, (d + m + 15) // 16, accum, 0)
        return 0

    lax.fori_loop(0, nchunks, chunk_body, 0)

    def finalize(r, _):
        c = cnt_ref[r + 1]
        denom = jnp.maximum(jnp.full((16,), c, jnp.float32), 1.0)
        off = (r + 1) * _DIM
        acc_ref[pl.ds(off, 16)] = acc_ref[pl.ds(off, 16)] / denom
        acc_ref[pl.ds(off + 16, 16)] = acc_ref[pl.ds(off + 16, 16)] / denom
        return 0

    lax.fori_loop(0, _SEG_PER_W, finalize, 0)

    pltpu.sync_copy(acc_ref.at[pl.ds(_DIM, _SEG_PER_W * _DIM)],
                    out_hbm.at[pl.ds(seg_base * _DIM, _SEG_PER_W * _DIM)])


@jax.jit
def _run(ids, segment_ids, params):
    table = _relayout(params)
    k = functools.partial(
        pl.kernel,
        out_type=jax.ShapeDtypeStruct((_BATCH * _DIM,), jnp.float32),
        mesh=plsc.VectorSubcoreMesh(core_axis_name="c", subcore_axis_name="s"),
        compiler_params=pltpu.CompilerParams(use_tc_tiling_on_sc=False),
        scratch_types=[
            pltpu.VMEM((16,), jnp.int32),
            pltpu.VMEM((_CHUNK,), jnp.int32),
            pltpu.VMEM((_CHUNK,), jnp.int32),
            pltpu.VMEM((_CHUNK, _DIM), jnp.float32),
            pltpu.VMEM((_ACC_ROWS * _DIM,), jnp.float32),
            pltpu.SMEM((_ACC_ROWS,), jnp.float32),
            pltpu.SemaphoreType.DMA,
        ],
    )(_gather_body)
    out = k(ids, segment_ids, table.reshape(_VPAD, _DIM))
    return out.reshape(_BATCH, _DIM)


def kernel(ids, segment_ids, params):
    return _run(ids, segment_ids, params)


# speedup vs baseline: 4.1792x; 1.0054x over previous
import functools

import jax
import jax.numpy as jnp
from jax import lax
from jax.experimental import pallas as pl
from jax.experimental.pallas import tpu as pltpu
from jax.experimental.pallas import tpu_sc as plsc

_VOCAB = 1000000
_DIM = 32
_BATCH = 4096
_NNZ = 204800

_NW = 32
_SEG_PER_W = _BATCH // _NW
_CHUNK = 1024
_SUB = 128
_ACC_ROWS = _SEG_PER_W + 2

_NCOL = 7813
_BB = 1344
_GB = 6
_NCOLP = _GB * _BB
_VPAD = _NCOL * 128


def _detile_body(in_ref, out_ref):
    x = in_ref[...]
    out_ref[...] = jnp.concatenate(
        [x[:, 128 * j:128 * (j + 1)] for j in range(_BB)], axis=0)


def _detile(params):
    return pl.pallas_call(
        _detile_body,
        grid=(4, _GB),
        in_specs=[pl.BlockSpec((8, _BB * 128), lambda a, g: (a, g))],
        out_specs=pl.BlockSpec((_BB * 8, 128), lambda a, g: (a * _GB + g, 0)),
        out_shape=jax.ShapeDtypeStruct((_NCOLP * 4 * 8, 128), jnp.float32),
    )(params.T)


def _shuffle_body(ts_hbm, out_hbm, tile_ref, row_ref, isem, osem):
    wid = lax.axis_index("c") * 16 + lax.axis_index("s")
    nblk = jnp.where(wid < _NCOL - 244 * _NW, 245, 244)
    lane = lax.broadcasted_iota(jnp.int32, (16,), 0)

    def fire_in(k, buf):
        b = wid + k * _NW
        for a in range(4):
            pltpu.make_async_copy(
                ts_hbm.at[pl.ds(pl.multiple_of((a * _NCOLP + b) * 8, 8), 8),
                          :],
                tile_ref.at[buf, pl.ds(8 * a, 8), pl.ds(0, 128)],
                isem).start()

    @pl.when(nblk > 0)
    def _():
        fire_in(0, 0)

    def block(k, _):
        b = wid + k * _NW
        buf = k % 2

        @pl.when(k + 1 < nblk)
        def _():
            fire_in(k + 1, 1 - buf)

        for a in range(4):
            pltpu.make_async_copy(
                ts_hbm.at[pl.ds(pl.multiple_of((a * _NCOLP + b) * 8, 8), 8),
                          :],
                tile_ref.at[buf, pl.ds(8 * a, 8), pl.ds(0, 128)],
                isem).wait()

        @pl.when(k >= 2)
        def _():
            b2 = wid + (k - 2) * _NW
            pltpu.make_async_copy(
                row_ref.at[buf],
                out_hbm.at[pl.ds(b2 * 128 * _DIM, 128 * _DIM)], osem).wait()

        def vgroup(g, _):
            for h in range(2):
                vs = [g * 16 + h * 8 + jj for jj in range(8)]
                los = [plsc.load_gather(
                    tile_ref.at[buf], [lane, jnp.full((16,), v, jnp.int32)])
                    for v in vs]
                his = [plsc.load_gather(
                    tile_ref.at[buf],
                    [lane + 16, jnp.full((16,), v, jnp.int32)])
                    for v in vs]
                for jj, v in enumerate(vs):
                    row_ref[buf, pl.ds(v * _DIM, 16)] = los[jj]
                    row_ref[buf, pl.ds(v * _DIM + 16, 16)] = his[jj]
            return 0

        lax.fori_loop(0, 8, vgroup, 0)
        pltpu.make_async_copy(
            row_ref.at[buf],
            out_hbm.at[pl.ds(b * 128 * _DIM, 128 * _DIM)], osem).start()
        return 0

    lax.fori_loop(0, nblk, block, 0)

    def drain(k, _):
        @pl.when(k >= jnp.maximum(nblk - 2, 0))
        def _():
            b2 = wid + k * _NW
            pltpu.make_async_copy(
                row_ref.at[k % 2],
                out_hbm.at[pl.ds(b2 * 128 * _DIM, 128 * _DIM)], osem).wait()
        return 0

    lax.fori_loop(0, nblk, drain, 0)


def _shuffle(ts):
    k = functools.partial(
        pl.kernel,
        out_type=jax.ShapeDtypeStruct((_VPAD * _DIM,), jnp.float32),
        mesh=plsc.VectorSubcoreMesh(core_axis_name="c", subcore_axis_name="s"),
        compiler_params=pltpu.CompilerParams(use_tc_tiling_on_sc=False,
                                             needs_layout_passes=False),
        scratch_types=[
            pltpu.VMEM((2, 32, 129), jnp.float32),
            pltpu.VMEM((2, 128 * _DIM), jnp.float32),
            pltpu.SemaphoreType.DMA,
            pltpu.SemaphoreType.DMA,
        ],
    )(_shuffle_body)
    return k(ts)


@jax.jit
def _relayout(params):
    return _shuffle(_detile(params))


def _sel16(v, k):
    s = v[0]
    for j in range(1, 16):
        s = jnp.where(k == j, v[j], s)
    return s


def _lower_bound(seg_hbm, probe_ref, target):
    def body(_, carry):
        lo, hi = carry
        m = (lo + hi) // 2
        m8 = pl.multiple_of(jnp.minimum(m & ~7, _NNZ - 16), 8)
        pltpu.sync_copy(seg_hbm.at[pl.ds(m8, 16)], probe_ref)
        v = _sel16(probe_ref[pl.ds(0, 16)], m - m8)
        lt = v < target
        lo = jnp.where(lt, m + 1, lo)
        hi = jnp.where(lt, hi, m)
        return lo, hi

    lo, _ = lax.fori_loop(0, 18, body, (jnp.int32(0), jnp.int32(_NNZ)))
    return lo


def _gather_body(ids_hbm, seg_hbm, table_hbm, out_hbm,
                 probe_ref, idx_ref, segv_ref, rows_ref,
                 acc_ref, cnt_ref, sem):
    wid = lax.axis_index("c") * 16 + lax.axis_index("s")
    seg_base = wid * _SEG_PER_W
    lane = lax.broadcasted_iota(jnp.int32, (16,), 0)

    def zero_acc(k, _):
        acc_ref[pl.ds(k * 16, 16)] = jnp.zeros((16,), jnp.float32)
        return 0

    lax.fori_loop(0, (_ACC_ROWS * _DIM) // 16, zero_acc, 0)

    def zero_cnt(k, _):
        cnt_ref[k] = 0.0
        return 0

    lax.fori_loop(0, _ACC_ROWS, zero_cnt, 0)

    start = _lower_bound(seg_hbm, probe_ref, seg_base)
    end = _lower_bound(seg_hbm, probe_ref, seg_base + _SEG_PER_W)

    a0 = start & ~7
    e8 = (end + 7) & ~7
    nchunks = (e8 - a0 + _CHUNK - 1) // _CHUNK

    def chunk_body(t, _):
        logical = a0 + t * _CHUNK
        p = pl.multiple_of(
            jnp.minimum(logical, _NNZ - _CHUNK), 8)
        d = logical - p
        m = jnp.minimum(_CHUNK, e8 - logical)

        pltpu.sync_copy(ids_hbm.at[pl.ds(p, _CHUNK)], idx_ref)
        pltpu.sync_copy(seg_hbm.at[pl.ds(p, _CHUNK)], segv_ref)

        copies = []
        for j in range(_CHUNK // _SUB):
            copies.append(pltpu.make_async_copy(
                table_hbm.at[idx_ref.at[pl.ds(j * _SUB, _SUB)]],
                rows_ref.at[pl.ds(j * _SUB, _SUB), :],
                sem,
            ))
        for c in copies:
            c.start()
        for c in copies:
            c.wait()

        def accum(g, _):
            base = pl.multiple_of(g * 16, 16)
            sv = segv_ref[pl.ds(base, 16)]
            pos = base + lane
            ok = (pos >= d) & (pos < d + m)
            rv = jnp.clip(jnp.where(ok, sv - seg_base, -1), -1, _SEG_PER_W) + 1
            offv = rv * _DIM
            for j in range(16):
                off = offv[j]
                acc_ref[pl.ds(off, 16)] = (
                    acc_ref[pl.ds(off, 16)] + rows_ref[base + j, pl.ds(0, 16)])
                acc_ref[pl.ds(off + 16, 16)] = (
                    acc_ref[pl.ds(off + 16, 16)]
                    + rows_ref[base + j, pl.ds(16, 16)])
                r = rv[j]
                cnt_ref[r] = cnt_ref[r] + 1.0
            return 0

        lax.fori_loop(d // 16, (d + m + 15) // 16, accum, 0)
        return 0

    lax.fori_loop(0, nchunks, chunk_body, 0)

    def finalize(r, _):
        c = cnt_ref[r + 1]
        denom = jnp.maximum(jnp.full((16,), c, jnp.float32), 1.0)
        off = (r + 1) * _DIM
        acc_ref[pl.ds(off, 16)] = acc_ref[pl.ds(off, 16)] / denom
        acc_ref[pl.ds(off + 16, 16)] = acc_ref[pl.ds(off + 16, 16)] / denom
        return 0

    lax.fori_loop(0, _SEG_PER_W, finalize, 0)

    pltpu.sync_copy(acc_ref.at[pl.ds(_DIM, _SEG_PER_W * _DIM)],
                    out_hbm.at[pl.ds(seg_base * _DIM, _SEG_PER_W * _DIM)])


@jax.jit
def _run(ids, segment_ids, params):
    table = _relayout(params)
    k = functools.partial(
        pl.kernel,
        out_type=jax.ShapeDtypeStruct((_BATCH * _DIM,), jnp.float32),
        mesh=plsc.VectorSubcoreMesh(core_axis_name="c", subcore_axis_name="s"),
        compiler_params=pltpu.CompilerParams(use_tc_tiling_on_sc=False),
        scratch_types=[
            pltpu.VMEM((16,), jnp.int32),
            pltpu.VMEM((_CHUNK,), jnp.int32),
            pltpu.VMEM((_CHUNK,), jnp.int32),
            pltpu.VMEM((_CHUNK, _DIM), jnp.float32),
            pltpu.VMEM((_ACC_ROWS * _DIM,), jnp.float32),
            pltpu.SMEM((_ACC_ROWS,), jnp.float32),
            pltpu.SemaphoreType.DMA,
        ],
    )(_gather_body)
    out = k(ids, segment_ids, table.reshape(_VPAD, _DIM))
    return out.reshape(_BATCH, _DIM)


def kernel(ids, segment_ids, params):
    return _run(ids, segment_ids, params)


# speedup vs baseline: 4.3285x; 1.0357x over previous
import functools

import jax
import jax.numpy as jnp
from jax import lax
from jax.experimental import pallas as pl
from jax.experimental.pallas import tpu as pltpu
from jax.experimental.pallas import tpu_sc as plsc

_VOCAB = 1000000
_DIM = 32
_BATCH = 4096
_NNZ = 204800

_NW = 32
_SEG_PER_W = _BATCH // _NW
_CHUNK = 1024
_SUB = 128
_ACC_ROWS = _SEG_PER_W + 2

_NCOL = 7813
_BB = 1344
_GB = 6
_NCOLP = _GB * _BB
_VPAD = _NCOL * 128


def _detile_body(in_ref, out_ref):
    x = in_ref[...]
    out_ref[...] = jnp.concatenate(
        [x[:, 128 * j:128 * (j + 1)] for j in range(_BB)], axis=0)


def _detile(params):
    return pl.pallas_call(
        _detile_body,
        grid=(4, _GB),
        in_specs=[pl.BlockSpec((8, _BB * 128), lambda a, g: (a, g))],
        out_specs=pl.BlockSpec((_BB * 8, 128), lambda a, g: (a * _GB + g, 0)),
        out_shape=jax.ShapeDtypeStruct((_NCOLP * 4 * 8, 128), jnp.float32),
    )(params.T)


def _shuffle_body(ts_hbm, out_hbm, tile_ref, row_ref, isem, osem):
    wid = lax.axis_index("c") * 16 + lax.axis_index("s")
    nblk = jnp.where(wid < _NCOL - 244 * _NW, 245, 244)
    lane = lax.broadcasted_iota(jnp.int32, (16,), 0)

    def fire_in(k, buf):
        b = wid + k * _NW
        for a in range(4):
            pltpu.make_async_copy(
                ts_hbm.at[pl.ds(pl.multiple_of((a * _NCOLP + b) * 8, 8), 8),
                          :],
                tile_ref.at[buf, pl.ds(8 * a, 8), pl.ds(0, 128)],
                isem).start()

    @pl.when(nblk > 0)
    def _():
        fire_in(0, 0)

    def block(k, _):
        b = wid + k * _NW
        buf = k % 2

        @pl.when(k + 1 < nblk)
        def _():
            fire_in(k + 1, 1 - buf)

        for a in range(4):
            pltpu.make_async_copy(
                ts_hbm.at[pl.ds(pl.multiple_of((a * _NCOLP + b) * 8, 8), 8),
                          :],
                tile_ref.at[buf, pl.ds(8 * a, 8), pl.ds(0, 128)],
                isem).wait()

        @pl.when(k >= 2)
        def _():
            b2 = wid + (k - 2) * _NW
            pltpu.make_async_copy(
                row_ref.at[buf],
                out_hbm.at[pl.ds(b2 * 128 * _DIM, 128 * _DIM)], osem).wait()

        def vgroup(g, _):
            for h in range(2):
                vs = [g * 16 + h * 8 + jj for jj in range(8)]
                los = [plsc.load_gather(
                    tile_ref.at[buf], [lane, jnp.full((16,), v, jnp.int32)])
                    for v in vs]
                his = [plsc.load_gather(
                    tile_ref.at[buf],
                    [lane + 16, jnp.full((16,), v, jnp.int32)])
                    for v in vs]
                for jj, v in enumerate(vs):
                    row_ref[buf, pl.ds(v * _DIM, 16)] = los[jj]
                    row_ref[buf, pl.ds(v * _DIM + 16, 16)] = his[jj]
            return 0

        lax.fori_loop(0, 8, vgroup, 0)
        pltpu.make_async_copy(
            row_ref.at[buf],
            out_hbm.at[pl.ds(b * 128 * _DIM, 128 * _DIM)], osem).start()
        return 0

    lax.fori_loop(0, nblk, block, 0)

    def drain(k, _):
        @pl.when(k >= jnp.maximum(nblk - 2, 0))
        def _():
            b2 = wid + k * _NW
            pltpu.make_async_copy(
                row_ref.at[k % 2],
                out_hbm.at[pl.ds(b2 * 128 * _DIM, 128 * _DIM)], osem).wait()
        return 0

    lax.fori_loop(0, nblk, drain, 0)


def _shuffle(ts):
    k = functools.partial(
        pl.kernel,
        out_type=jax.ShapeDtypeStruct((_VPAD * _DIM,), jnp.float32),
        mesh=plsc.VectorSubcoreMesh(core_axis_name="c", subcore_axis_name="s"),
        compiler_params=pltpu.CompilerParams(use_tc_tiling_on_sc=False,
                                             needs_layout_passes=False),
        scratch_types=[
            pltpu.VMEM((2, 32, 129), jnp.float32),
            pltpu.VMEM((2, 128 * _DIM), jnp.float32),
            pltpu.SemaphoreType.DMA,
            pltpu.SemaphoreType.DMA,
        ],
    )(_shuffle_body)
    return k(ts)


@jax.jit
def _relayout(params):
    return _shuffle(_detile(params))


def _sel16(v, k):
    s = v[0]
    for j in range(1, 16):
        s = jnp.where(k == j, v[j], s)
    return s


def _lower_bound(seg_hbm, probe_ref, target):
    def body(_, carry):
        lo, hi = carry
        m = (lo + hi) // 2
        m8 = pl.multiple_of(jnp.minimum(m & ~7, _NNZ - 16), 8)
        pltpu.sync_copy(seg_hbm.at[pl.ds(m8, 16)], probe_ref)
        v = _sel16(probe_ref[pl.ds(0, 16)], m - m8)
        lt = v < target
        lo = jnp.where(lt, m + 1, lo)
        hi = jnp.where(lt, hi, m)
        return lo, hi

    lo, _ = lax.fori_loop(0, 18, body, (jnp.int32(0), jnp.int32(_NNZ)))
    return lo


def _gather_body(ids_hbm, seg_hbm, table_hbm, out_hbm,
                 probe_ref, idx_ref, segv_ref, rows_ref,
                 acc_ref, cnt_ref, sem):
    wid = lax.axis_index("c") * 16 + lax.axis_index("s")
    seg_base = wid * _SEG_PER_W
    lane = lax.broadcasted_iota(jnp.int32, (16,), 0)

    def zero_acc(k, _):
        acc_ref[pl.ds(k * 16, 16)] = jnp.zeros((16,), jnp.float32)
        return 0

    lax.fori_loop(0, (_ACC_ROWS * _DIM) // 16, zero_acc, 0)

    def zero_cnt(k, _):
        cnt_ref[k] = 0.0
        return 0

    lax.fori_loop(0, _ACC_ROWS, zero_cnt, 0)

    start = _lower_bound(seg_hbm, probe_ref, seg_base)
    end = _lower_bound(seg_hbm, probe_ref, seg_base + _SEG_PER_W)

    a0 = start & ~7
    e8 = (end + 7) & ~7
    nchunks = (e8 - a0 + _CHUNK - 1) // _CHUNK

    def stage(t, buf):
        logical = a0 + t * _CHUNK
        p = pl.multiple_of(
            jnp.minimum(logical, _NNZ - _CHUNK), 8)
        pltpu.sync_copy(ids_hbm.at[pl.ds(p, _CHUNK)], idx_ref.at[buf])
        pltpu.sync_copy(seg_hbm.at[pl.ds(p, _CHUNK)], segv_ref.at[buf])
        for j in range(_CHUNK // _SUB):
            pltpu.make_async_copy(
                table_hbm.at[idx_ref.at[buf, pl.ds(j * _SUB, _SUB)]],
                rows_ref.at[buf, pl.ds(j * _SUB, _SUB), :],
                sem,
            ).start()

    @pl.when(nchunks > 0)
    def _():
        stage(0, 0)

    def chunk_body(t, _):
        buf = t % 2
        logical = a0 + t * _CHUNK
        d = logical - pl.multiple_of(
            jnp.minimum(logical, _NNZ - _CHUNK), 8)
        m = jnp.minimum(_CHUNK, e8 - logical)

        for j in range(_CHUNK // _SUB):
            pltpu.make_async_copy(
                table_hbm.at[idx_ref.at[buf, pl.ds(j * _SUB, _SUB)]],
                rows_ref.at[buf, pl.ds(j * _SUB, _SUB), :],
                sem,
            ).wait()

        @pl.when(t + 1 < nchunks)
        def _():
            stage(t + 1, 1 - buf)

        def accum(g, _):
            base = pl.multiple_of(g * 16, 16)
            sv = segv_ref[buf, pl.ds(base, 16)]
            pos = base + lane
            ok = (pos >= d) & (pos < d + m)
            rv = jnp.clip(jnp.where(ok, sv - seg_base, -1), -1, _SEG_PER_W) + 1
            offv = rv * _DIM
            for j in range(16):
                off = offv[j]
                acc_ref[pl.ds(off, 16)] = (
                    acc_ref[pl.ds(off, 16)]
                    + rows_ref[buf, base + j, pl.ds(0, 16)])
                acc_ref[pl.ds(off + 16, 16)] = (
                    acc_ref[pl.ds(off + 16, 16)]
                    + rows_ref[buf, base + j, pl.ds(16, 16)])
                r = rv[j]
                cnt_ref[r] = cnt_ref[r] + 1.0
            return 0

        lax.fori_loop(d // 16, (d + m + 15) // 16, accum, 0)
        return 0

    lax.fori_loop(0, nchunks, chunk_body, 0)

    def finalize(r, _):
        c = cnt_ref[r + 1]
        denom = jnp.maximum(jnp.full((16,), c, jnp.float32), 1.0)
        off = (r + 1) * _DIM
        acc_ref[pl.ds(off, 16)] = acc_ref[pl.ds(off, 16)] / denom
        acc_ref[pl.ds(off + 16, 16)] = acc_ref[pl.ds(off + 16, 16)] / denom
        return 0

    lax.fori_loop(0, _SEG_PER_W, finalize, 0)

    pltpu.sync_copy(acc_ref.at[pl.ds(_DIM, _SEG_PER_W * _DIM)],
                    out_hbm.at[pl.ds(seg_base * _DIM, _SEG_PER_W * _DIM)])


@jax.jit
def _run(ids, segment_ids, params):
    table = _relayout(params)
    k = functools.partial(
        pl.kernel,
        out_type=jax.ShapeDtypeStruct((_BATCH * _DIM,), jnp.float32),
        mesh=plsc.VectorSubcoreMesh(core_axis_name="c", subcore_axis_name="s"),
        compiler_params=pltpu.CompilerParams(use_tc_tiling_on_sc=False),
        scratch_types=[
            pltpu.VMEM((16,), jnp.int32),
            pltpu.VMEM((2, _CHUNK), jnp.int32),
            pltpu.VMEM((2, _CHUNK), jnp.int32),
            pltpu.VMEM((2, _CHUNK, _DIM), jnp.float32),
            pltpu.VMEM((_ACC_ROWS * _DIM,), jnp.float32),
            pltpu.SMEM((_ACC_ROWS,), jnp.float32),
            pltpu.SemaphoreType.DMA,
        ],
    )(_gather_body)
    out = k(ids, segment_ids, table.reshape(_VPAD, _DIM))
    return out.reshape(_BATCH, _DIM)


def kernel(ids, segment_ids, params):
    return _run(ids, segment_ids, params)
